# baseline (speedup 1.0000x reference)
import functools

import jax
import jax.numpy as jnp
from jax import lax
from jax.experimental import pallas as pl
from jax.experimental.pallas import tpu as pltpu
from jax.experimental.pallas import tpu_sc as plsc

NC = 2
NS = 16
NW = NC * NS
D = 128
NBUF = 8


def _build(b, s):
    rows_w = b // NW
    assert rows_w * NW == b and rows_w % NBUF == 0

    mesh = plsc.VectorSubcoreMesh(core_axis_name="c", subcore_axis_name="s")

    @functools.partial(
        pl.kernel,
        out_type=jax.ShapeDtypeStruct((b, s, D), jnp.float32),
        mesh=mesh,
        compiler_params=pltpu.CompilerParams(use_tc_tiling_on_sc=True),
        scratch_types=[
            pltpu.VMEM((rows_w, s), jnp.int32),
            pltpu.VMEM((NBUF, s, D), jnp.float32),
        ] + [pltpu.SemaphoreType.DMA] * NBUF,
    )
    def gather_kernel(idx_hbm, table_hbm, out_hbm, idx_v, rows_v, *sems):
        wid = lax.axis_index("s") * NC + lax.axis_index("c")
        row0 = wid * rows_w
        pltpu.sync_copy(idx_hbm.at[pl.ds(row0, rows_w)], idx_v)

        def start(g, buf):
            pltpu.async_copy(table_hbm.at[idx_v.at[g]], rows_v.at[buf], sems[buf])

        def finish(g, buf):
            pltpu.make_async_copy(
                table_hbm.at[idx_v.at[g]], rows_v.at[buf], sems[buf]
            ).wait()

        for buf in range(NBUF):
            start(buf, buf)

        def group(i, carry):
            g0 = i * NBUF
            for buf in range(NBUF):
                g = g0 + buf
                finish(g, buf)
                pltpu.sync_copy(rows_v.at[buf], out_hbm.at[row0 + g])
                nxt = g + NBUF

                @pl.when(nxt < rows_w)
                def _():
                    start(nxt, buf)
            return carry

        lax.fori_loop(0, rows_w // NBUF, group, None)

    return gather_kernel


@jax.jit
def kernel(x, table):
    b, s = x.shape
    return _build(b, s)(x, table)


# speedup vs baseline: 1.8100x; 1.8100x over previous
import functools

import jax
import jax.numpy as jnp
from jax import lax
from jax.experimental import pallas as pl
from jax.experimental.pallas import tpu as pltpu
from jax.experimental.pallas import tpu_sc as plsc

NC = 2
NS = 16
NW = NC * NS
D = 128
BW = 128
NBUF = 5


def _build(b, s):
    assert b == NW * BW and s % NBUF == 0

    mesh = plsc.VectorSubcoreMesh(core_axis_name="c", subcore_axis_name="s")

    @functools.partial(
        pl.kernel,
        out_type=jax.ShapeDtypeStruct((s, b, D), jnp.float32),
        mesh=mesh,
        scratch_types=[
            pltpu.VMEM((s, BW), jnp.int32),
            pltpu.VMEM((NBUF, BW, D), jnp.float32),
        ] + [pltpu.SemaphoreType.DMA] * NBUF,
    )
    def gather_kernel(idx_hbm, table_hbm, out_hbm, idx_v, rows_v, *sems):
        wid = lax.axis_index("s") * NC + lax.axis_index("c")
        col0 = wid * BW
        pltpu.sync_copy(idx_hbm.at[:, pl.ds(col0, BW)], idx_v)

        def start(g, buf):
            pltpu.async_copy(table_hbm.at[idx_v.at[g]], rows_v.at[buf], sems[buf])

        def finish(g, buf):
            pltpu.make_async_copy(
                table_hbm.at[idx_v.at[g]], rows_v.at[buf], sems[buf]
            ).wait()

        for buf in range(NBUF):
            start(buf, buf)

        def group(i, carry):
            g0 = i * NBUF
            for buf in range(NBUF):
                g = g0 + buf
                finish(g, buf)
                pltpu.sync_copy(rows_v.at[buf], out_hbm.at[g, pl.ds(col0, BW)])
                nxt = g + NBUF

                @pl.when(nxt < s)
                def _():
                    start(nxt, buf)
            return carry

        lax.fori_loop(0, s // NBUF, group, None)

    return gather_kernel


@jax.jit
def kernel(x, table):
    b, s = x.shape
    p = _build(b, s)(x.T, table)
    return jnp.transpose(p, (1, 0, 2))
